# baseline (speedup 1.0000x reference)
import functools

import jax
import jax.numpy as jnp
from jax import lax
from jax.experimental import pallas as pl
from jax.experimental.pallas import tpu as pltpu
from jax.experimental.pallas import tpu_sc as plsc

_STRIDES = (8, 16, 32, 64, 128)
_LIMITS = ((-1.0, 64.0), (64.0, 128.0), (128.0, 256.0), (256.0, 512.0),
           (512.0, 999999.0))
_IMG = 1024
_B = 4
_M = 64
_HS = tuple(_IMG // s for s in _STRIDES)
_HW = sum(h * h for h in _HS)
_LVL_BASE = tuple(sum(h * h for h in _HS[:i]) for i in range(5))
_NC = 2
_NS = 16
_NW = _NC * _NS
_WPB = _NW // _B
_P = 22016
_BASES = (0, 3280, 6560, 9840, 13120, 16384, 20480, 21504)
_SIZES = (3280, 3280, 3280, 3280, 3264, 4096, 1024, 512)
_BUF = max(_SIZES)
_SENTINEL = 99999999.0


@functools.cache
def _build_sc_targets():
    mesh = plsc.VectorSubcoreMesh(core_axis_name="c", subcore_axis_name="s")
    return pl.kernel(
        _sc_targets_body,
        mesh=mesh,
        compiler_params=pltpu.CompilerParams(needs_layout_passes=False),
        out_type=[
            jax.ShapeDtypeStruct((_B * _P,), jnp.int32),
            jax.ShapeDtypeStruct((_B * _P,), jnp.float32),
            jax.ShapeDtypeStruct((_B * _P,), jnp.float32),
            jax.ShapeDtypeStruct((_B * _P,), jnp.float32),
            jax.ShapeDtypeStruct((_B * _P,), jnp.float32),
            jax.ShapeDtypeStruct((_B * _P,), jnp.float32),
        ],
        scratch_types=[
            pltpu.VMEM((_M * 5 * 16,), jnp.int32),
            pltpu.VMEM((_BUF,), jnp.float32),
            pltpu.VMEM((_BUF,), jnp.int32),
            pltpu.VMEM((_BUF,), jnp.float32),
            pltpu.VMEM((_BUF,), jnp.float32),
            pltpu.VMEM((_BUF,), jnp.float32),
            pltpu.VMEM((_BUF,), jnp.float32),
            pltpu.SemaphoreType.DMA,
        ],
    )


def _sc_targets_body(packed_h,
                     cls_o, cen_o, l_o, t_o, r_o, b_o,
                     pk_v, area_b, lab_b, l_b, t_b, r_b, b_b, sem):
    w = lax.axis_index("s") * _NC + lax.axis_index("c")
    bat = w // _WPB
    k = w % _WPB
    base = jnp.int32(_BASES[0])
    size = jnp.int32(_SIZES[0])
    for i in range(1, _WPB):
        base = jnp.where(k == i, _BASES[i], base)
        size = jnp.where(k == i, _SIZES[i], size)
    nv = size // 16

    pltpu.sync_copy(packed_h.at[pl.ds(bat * (_M * 5 * 16), _M * 5 * 16)], pk_v)

    sentinel = jnp.full((16,), _SENTINEL, jnp.float32)
    zero = jnp.full((16,), 0.0, jnp.float32)
    one = jnp.full((16,), 1.0, jnp.float32)
    lane = lax.iota(jnp.int32, 16)
    dx = lane & 3
    dy = lane >> 2

    def init_body(i, carry):
        area_b[pl.ds(i * 16, 16)] = sentinel
        return carry

    lax.fori_loop(0, nv, init_body, 0)

    for lv in range(5):
        s = float(_STRIDES[lv])
        inv_s = 1.0 / s
        h = _HS[lv]
        lo = float(_LIMITS[lv][0])
        hi = float(_LIMITS[lv][1])
        rad = 1.5 * s
        lvl_lo = _LVL_BASE[lv]
        lvl_hi = lvl_lo + h * h
        nb = lvl_lo - base

        def box_body(j, carry, inv_s=inv_s, s=s, h=h, lo=lo, hi=hi, rad=rad,
                     nb=nb):
            q = j * 80
            f32 = functools.partial(lax.bitcast_convert_type,
                                    new_dtype=jnp.float32)
            x1 = f32(pk_v[pl.ds(q, 16)])
            y1 = f32(pk_v[pl.ds(q + 16, 16)])
            x2 = f32(pk_v[pl.ds(q + 32, 16)])
            y2 = f32(pk_v[pl.ds(q + 48, 16)])
            labj = pk_v[pl.ds(q + 64, 16)]
            cx = (x1 + x2) * 0.5
            cy = (y1 + y2) * 0.5
            kx = (cx * inv_s - 0.5).astype(jnp.int32) - 1 + dx
            ky = (cy * inv_s - 0.5).astype(jnp.int32) - 1 + dy
            xw = (kx.astype(jnp.float32) + 0.5) * s
            yw = (ky.astype(jnp.float32) + 0.5) * s
            lft = xw - x1
            top = yw - y1
            rgt = x2 - xw
            bot = y2 - yw
            area = (lft + rgt) * (top + bot)
            omin = jnp.minimum(jnp.minimum(lft, top), jnp.minimum(rgt, bot))
            omax = jnp.maximum(jnp.maximum(lft, top), jnp.maximum(rgt, bot))
            cd = jnp.maximum(jnp.abs(xw - cx), jnp.abs(yw - cy))
            m = (omin > zero) & (omax > lo) & (omax <= hi) & (cd < rad)
            ploc = ky * h + kx + nb
            own = m & (ploc >= 0) & (ploc < size)
            idx = jnp.clip(ploc, 0, _BUF - 1)
            cur = plsc.load_gather(area_b, [idx], mask=own)
            upd = own & (area < cur)
            plsc.store_scatter(area_b, [idx], area, mask=upd)
            plsc.store_scatter(l_b, [idx], lft, mask=upd)
            plsc.store_scatter(t_b, [idx], top, mask=upd)
            plsc.store_scatter(r_b, [idx], rgt, mask=upd)
            plsc.store_scatter(b_b, [idx], bot, mask=upd)
            plsc.store_scatter(lab_b, [idx], labj, mask=upd)
            return carry

        def box_quad(jj, carry, box_body=box_body):
            for u in range(4):
                box_body(jj * 4 + u, 0)
            return carry

        @pl.when((lvl_hi > base) & (lvl_lo < base + size))
        def _(box_quad=box_quad):
            lax.fori_loop(0, _M // 4, box_quad, 0)

    neg1 = jnp.full((16,), -1.0, jnp.float32)
    izero = jnp.full((16,), 0, jnp.int32)
    thresh = jnp.full((16,), 9.0e7, jnp.float32)
    magic = jnp.full((16,), 0x1FBD1DF5, jnp.int32)

    def fin_body(i, carry):
        o = i * 16
        av = area_b[pl.ds(o, 16)]
        bl = l_b[pl.ds(o, 16)]
        bt = t_b[pl.ds(o, 16)]
        br = r_b[pl.ds(o, 16)]
        bb = b_b[pl.ds(o, 16)]
        blab = lab_b[pl.ds(o, 16)]
        posm = av < thresh
        lrmin = jnp.minimum(bl, br)
        lrmax = jnp.maximum(bl, br)
        tbmin = jnp.minimum(bt, bb)
        tbmax = jnp.maximum(bt, bb)
        ratio = lrmin * tbmin / (lrmax * tbmax + 1e-10)
        rs = jnp.where(posm, ratio, one)
        sq = lax.bitcast_convert_type(
            (lax.bitcast_convert_type(rs, jnp.int32) >> 1) + magic,
            jnp.float32)
        sq = (sq + rs / sq) * 0.5
        sq = (sq + rs / sq) * 0.5
        area_b[pl.ds(o, 16)] = jnp.where(posm, sq, neg1)
        lab_b[pl.ds(o, 16)] = jnp.where(posm, blab, izero)
        l_b[pl.ds(o, 16)] = jnp.where(posm, bl, neg1)
        t_b[pl.ds(o, 16)] = jnp.where(posm, bt, neg1)
        r_b[pl.ds(o, 16)] = jnp.where(posm, br, neg1)
        b_b[pl.ds(o, 16)] = jnp.where(posm, bb, neg1)
        return carry

    lax.fori_loop(0, nv, fin_body, 0)

    off = pl.multiple_of(bat * _P + base, 16)
    pairs = ((lab_b, cls_o), (area_b, cen_o), (l_b, l_o),
             (t_b, t_o), (r_b, r_o), (b_b, b_o))
    for sz, cond in ((3280, k < 4), (3264, k == 4), (4096, k == 5),
                     (1024, k == 6), (512, k == 7)):
        @pl.when(cond)
        def _(sz=sz):
            handles = [
                pltpu.async_copy(src.at[pl.ds(0, sz)],
                                 dst.at[pl.ds(off, sz)], sem)
                for src, dst in pairs
            ]
            for hd in handles:
                hd.wait()


def kernel(cls_logit_0, center_logit_0, reg_logit_0,
           cls_logit_1, center_logit_1, reg_logit_1,
           cls_logit_2, center_logit_2, reg_logit_2,
           cls_logit_3, center_logit_3, reg_logit_3,
           cls_logit_4, center_logit_4, reg_logit_4,
           gt_box, labels):
    packed = jnp.concatenate([
        lax.bitcast_convert_type(gt_box.astype(jnp.float32), jnp.int32),
        labels.astype(jnp.int32)[..., None],
    ], axis=-1)
    packed_bc = jnp.broadcast_to(packed[..., None], (_B, _M, 5, 16)).reshape(-1)
    cls_f, cen_f, l_f, t_f, r_f, b_f = _build_sc_targets()(packed_bc)
    cls_t = cls_f.reshape(_B, _P)[:, :_HW, None]
    cen_t = cen_f.reshape(_B, _P)[:, :_HW, None]
    reg_t = jnp.stack(
        [a.reshape(_B, _P)[:, :_HW] for a in (l_f, t_f, r_f, b_f)], axis=-1)
    return cls_t, cen_t, reg_t


# speedup vs baseline: 1.0344x; 1.0344x over previous
import functools

import jax
import jax.numpy as jnp
from jax import lax
from jax.experimental import pallas as pl
from jax.experimental.pallas import tpu as pltpu
from jax.experimental.pallas import tpu_sc as plsc

_STRIDES = (8, 16, 32, 64, 128)
_LIMITS = ((-1.0, 64.0), (64.0, 128.0), (128.0, 256.0), (256.0, 512.0),
           (512.0, 999999.0))
_IMG = 1024
_B = 4
_M = 64
_HS = tuple(_IMG // s for s in _STRIDES)
_HW = sum(h * h for h in _HS)
_LVL_BASE = tuple(sum(h * h for h in _HS[:i]) for i in range(5))
_NC = 2
_NS = 16
_NW = _NC * _NS
_WPB = _NW // _B
_P = 22016
_BASES = (0, 3280, 6560, 9840, 13120, 16384, 20480, 21504)
_SIZES = (3280, 3280, 3280, 3280, 3264, 4096, 1024, 512)
_BUF = max(_SIZES)
_SENTINEL = 99999999.0


@functools.cache
def _build_sc_targets():
    mesh = plsc.VectorSubcoreMesh(core_axis_name="c", subcore_axis_name="s")
    return pl.kernel(
        _sc_targets_body,
        mesh=mesh,
        compiler_params=pltpu.CompilerParams(needs_layout_passes=False),
        out_type=[
            jax.ShapeDtypeStruct((_B * _P,), jnp.int32),
            jax.ShapeDtypeStruct((_B * _P,), jnp.float32),
            jax.ShapeDtypeStruct((_B * _P,), jnp.float32),
            jax.ShapeDtypeStruct((_B * _P,), jnp.float32),
            jax.ShapeDtypeStruct((_B * _P,), jnp.float32),
            jax.ShapeDtypeStruct((_B * _P,), jnp.float32),
        ],
        scratch_types=[
            pltpu.VMEM((_M * 5 * 16,), jnp.int32),
            pltpu.VMEM((_BUF,), jnp.float32),
            pltpu.VMEM((_BUF,), jnp.int32),
            pltpu.VMEM((_BUF,), jnp.float32),
            pltpu.VMEM((_BUF,), jnp.float32),
            pltpu.VMEM((_BUF,), jnp.float32),
            pltpu.VMEM((_BUF,), jnp.float32),
            pltpu.SemaphoreType.DMA,
        ],
    )


def _sc_targets_body(packed_h,
                     cls_o, cen_o, l_o, t_o, r_o, b_o,
                     pk_v, area_b, lab_b, l_b, t_b, r_b, b_b, sem):
    w = lax.axis_index("s") * _NC + lax.axis_index("c")
    bat = w // _WPB
    k = w % _WPB
    base = jnp.int32(_BASES[0])
    size = jnp.int32(_SIZES[0])
    for i in range(1, _WPB):
        base = jnp.where(k == i, _BASES[i], base)
        size = jnp.where(k == i, _SIZES[i], size)
    nv = size // 16

    pltpu.sync_copy(packed_h.at[pl.ds(bat * (_M * 5 * 16), _M * 5 * 16)], pk_v)

    sentinel = jnp.full((16,), _SENTINEL, jnp.float32)
    zero = jnp.full((16,), 0.0, jnp.float32)
    one = jnp.full((16,), 1.0, jnp.float32)
    lane = lax.iota(jnp.int32, 16)
    dx = lane & 3
    dy = lane >> 2

    def init_body(i, carry):
        area_b[pl.ds(i * 16, 16)] = sentinel
        return carry

    lax.fori_loop(0, nv, init_body, 0)

    for lv in range(5):
        s = float(_STRIDES[lv])
        inv_s = 1.0 / s
        h = _HS[lv]
        lo = float(_LIMITS[lv][0])
        hi = float(_LIMITS[lv][1])
        rad = 1.5 * s
        lvl_lo = _LVL_BASE[lv]
        lvl_hi = lvl_lo + h * h
        nb = lvl_lo - base

        def box_body(j, carry, inv_s=inv_s, s=s, h=h, lo=lo, hi=hi, rad=rad,
                     nb=nb):
            q = j * 80
            f32 = functools.partial(lax.bitcast_convert_type,
                                    new_dtype=jnp.float32)
            x1 = f32(pk_v[pl.ds(q, 16)])
            y1 = f32(pk_v[pl.ds(q + 16, 16)])
            x2 = f32(pk_v[pl.ds(q + 32, 16)])
            y2 = f32(pk_v[pl.ds(q + 48, 16)])
            labj = pk_v[pl.ds(q + 64, 16)]
            cx = (x1 + x2) * 0.5
            cy = (y1 + y2) * 0.5
            kx = (cx * inv_s - 0.5).astype(jnp.int32) - 1 + dx
            ky = (cy * inv_s - 0.5).astype(jnp.int32) - 1 + dy
            xw = (kx.astype(jnp.float32) + 0.5) * s
            yw = (ky.astype(jnp.float32) + 0.5) * s
            lft = xw - x1
            top = yw - y1
            rgt = x2 - xw
            bot = y2 - yw
            area = (lft + rgt) * (top + bot)
            omin = jnp.minimum(jnp.minimum(lft, top), jnp.minimum(rgt, bot))
            omax = jnp.maximum(jnp.maximum(lft, top), jnp.maximum(rgt, bot))
            cd = jnp.maximum(jnp.abs(xw - cx), jnp.abs(yw - cy))
            m = (omin > zero) & (omax > lo) & (omax <= hi) & (cd < rad)
            ploc = ky * h + kx + nb
            own = m & (ploc >= 0) & (ploc < size)
            idx = jnp.clip(ploc, 0, _BUF - 1)
            cur = plsc.load_gather(area_b, [idx], mask=own)
            upd = own & (area < cur)
            plsc.store_scatter(area_b, [idx], area, mask=upd)
            plsc.store_scatter(l_b, [idx], lft, mask=upd)
            plsc.store_scatter(t_b, [idx], top, mask=upd)
            plsc.store_scatter(r_b, [idx], rgt, mask=upd)
            plsc.store_scatter(b_b, [idx], bot, mask=upd)
            plsc.store_scatter(lab_b, [idx], labj, mask=upd)
            return carry

        @pl.when((lvl_hi > base) & (lvl_lo < base + size))
        def _():
            lax.fori_loop(0, _M, box_body, 0)

    neg1 = jnp.full((16,), -1.0, jnp.float32)
    izero = jnp.full((16,), 0, jnp.int32)
    thresh = jnp.full((16,), 9.0e7, jnp.float32)
    magic = jnp.full((16,), 0x1FBD1DF5, jnp.int32)

    def fin_body(i, carry):
        o = i * 16
        av = area_b[pl.ds(o, 16)]
        bl = l_b[pl.ds(o, 16)]
        bt = t_b[pl.ds(o, 16)]
        br = r_b[pl.ds(o, 16)]
        bb = b_b[pl.ds(o, 16)]
        blab = lab_b[pl.ds(o, 16)]
        posm = av < thresh
        lrmin = jnp.minimum(bl, br)
        lrmax = jnp.maximum(bl, br)
        tbmin = jnp.minimum(bt, bb)
        tbmax = jnp.maximum(bt, bb)
        ratio = lrmin * tbmin / (lrmax * tbmax + 1e-10)
        rs = jnp.where(posm, ratio, one)
        sq = lax.bitcast_convert_type(
            (lax.bitcast_convert_type(rs, jnp.int32) >> 1) + magic,
            jnp.float32)
        sq = (sq + rs / sq) * 0.5
        sq = (sq + rs / sq) * 0.5
        area_b[pl.ds(o, 16)] = jnp.where(posm, sq, neg1)
        lab_b[pl.ds(o, 16)] = jnp.where(posm, blab, izero)
        l_b[pl.ds(o, 16)] = jnp.where(posm, bl, neg1)
        t_b[pl.ds(o, 16)] = jnp.where(posm, bt, neg1)
        r_b[pl.ds(o, 16)] = jnp.where(posm, br, neg1)
        b_b[pl.ds(o, 16)] = jnp.where(posm, bb, neg1)
        return carry

    lax.fori_loop(0, nv, fin_body, 0)

    off = pl.multiple_of(bat * _P + base, 16)
    pairs = ((lab_b, cls_o), (area_b, cen_o), (l_b, l_o),
             (t_b, t_o), (r_b, r_o), (b_b, b_o))
    for sz, cond in ((3280, k < 4), (3264, k == 4), (4096, k == 5),
                     (1024, k == 6), (512, k == 7)):
        @pl.when(cond)
        def _(sz=sz):
            handles = [
                pltpu.async_copy(src.at[pl.ds(0, sz)],
                                 dst.at[pl.ds(off, sz)], sem)
                for src, dst in pairs
            ]
            for hd in handles:
                hd.wait()


def kernel(cls_logit_0, center_logit_0, reg_logit_0,
           cls_logit_1, center_logit_1, reg_logit_1,
           cls_logit_2, center_logit_2, reg_logit_2,
           cls_logit_3, center_logit_3, reg_logit_3,
           cls_logit_4, center_logit_4, reg_logit_4,
           gt_box, labels):
    packed = jnp.concatenate([
        lax.bitcast_convert_type(gt_box.astype(jnp.float32), jnp.int32),
        labels.astype(jnp.int32)[..., None],
    ], axis=-1)
    packed_bc = jnp.broadcast_to(packed[..., None], (_B, _M, 5, 16)).reshape(-1)
    cls_f, cen_f, l_f, t_f, r_f, b_f = _build_sc_targets()(packed_bc)
    cls_t = cls_f.reshape(_B, _P)[:, :_HW, None]
    cen_t = cen_f.reshape(_B, _P)[:, :_HW, None]
    reg_t = jnp.stack(
        [a.reshape(_B, _P)[:, :_HW] for a in (l_f, t_f, r_f, b_f)], axis=-1)
    return cls_t, cen_t, reg_t


# speedup vs baseline: 1.0796x; 1.0436x over previous
import functools

import jax
import jax.numpy as jnp
from jax import lax
from jax.experimental import pallas as pl
from jax.experimental.pallas import tpu as pltpu
from jax.experimental.pallas import tpu_sc as plsc

_STRIDES = (8, 16, 32, 64, 128)
_LIMITS = ((-1.0, 64.0), (64.0, 128.0), (128.0, 256.0), (256.0, 512.0),
           (512.0, 999999.0))
_IMG = 1024
_B = 4
_M = 64
_HS = tuple(_IMG // s for s in _STRIDES)
_HW = sum(h * h for h in _HS)
_LVL_BASE = tuple(sum(h * h for h in _HS[:i]) for i in range(5))
_NC = 2
_NS = 16
_NW = _NC * _NS
_WPB = _NW // _B
_P = 22016
_BASES = (0, 3280, 6560, 9840, 13120, 16384, 18432, 20480)
_SIZES = (3280, 3280, 3280, 3280, 3264, 2048, 2048, 1536)
_BUF = max(_SIZES)
_SENTINEL = 99999999.0


@functools.cache
def _build_sc_targets():
    mesh = plsc.VectorSubcoreMesh(core_axis_name="c", subcore_axis_name="s")
    return pl.kernel(
        _sc_targets_body,
        mesh=mesh,
        compiler_params=pltpu.CompilerParams(needs_layout_passes=False),
        out_type=[
            jax.ShapeDtypeStruct((_B * _P,), jnp.int32),
            jax.ShapeDtypeStruct((_B * _P,), jnp.float32),
            jax.ShapeDtypeStruct((_B * _P,), jnp.float32),
            jax.ShapeDtypeStruct((_B * _P,), jnp.float32),
            jax.ShapeDtypeStruct((_B * _P,), jnp.float32),
            jax.ShapeDtypeStruct((_B * _P,), jnp.float32),
        ],
        scratch_types=[
            pltpu.VMEM((_M * 5 * 16,), jnp.int32),
            pltpu.VMEM((_BUF,), jnp.float32),
            pltpu.VMEM((_BUF,), jnp.int32),
            pltpu.VMEM((_BUF,), jnp.float32),
            pltpu.VMEM((_BUF,), jnp.float32),
            pltpu.VMEM((_BUF,), jnp.float32),
            pltpu.VMEM((_BUF,), jnp.float32),
            pltpu.SemaphoreType.DMA,
        ],
    )


def _sc_targets_body(packed_h,
                     cls_o, cen_o, l_o, t_o, r_o, b_o,
                     pk_v, area_b, lab_b, l_b, t_b, r_b, b_b, sem):
    w = lax.axis_index("s") * _NC + lax.axis_index("c")
    bat = w // _WPB
    k = w % _WPB
    base = jnp.int32(_BASES[0])
    size = jnp.int32(_SIZES[0])
    for i in range(1, _WPB):
        base = jnp.where(k == i, _BASES[i], base)
        size = jnp.where(k == i, _SIZES[i], size)
    nv = size // 16

    pltpu.sync_copy(packed_h.at[pl.ds(bat * (_M * 5 * 16), _M * 5 * 16)], pk_v)

    sentinel = jnp.full((16,), _SENTINEL, jnp.float32)
    zero = jnp.full((16,), 0.0, jnp.float32)
    one = jnp.full((16,), 1.0, jnp.float32)
    lane = lax.iota(jnp.int32, 16)
    dx = lane & 3
    dy = lane >> 2

    def init_body(i, carry):
        area_b[pl.ds(i * 16, 16)] = sentinel
        return carry

    lax.fori_loop(0, nv, init_body, 0)

    for lv in range(5):
        s = float(_STRIDES[lv])
        inv_s = 1.0 / s
        h = _HS[lv]
        lo = float(_LIMITS[lv][0])
        hi = float(_LIMITS[lv][1])
        rad = 1.5 * s
        lvl_lo = _LVL_BASE[lv]
        lvl_hi = lvl_lo + h * h
        nb = lvl_lo - base

        def box_body(j, carry, inv_s=inv_s, s=s, h=h, lo=lo, hi=hi, rad=rad,
                     nb=nb):
            q = j * 80
            f32 = functools.partial(lax.bitcast_convert_type,
                                    new_dtype=jnp.float32)
            x1 = f32(pk_v[pl.ds(q, 16)])
            y1 = f32(pk_v[pl.ds(q + 16, 16)])
            x2 = f32(pk_v[pl.ds(q + 32, 16)])
            y2 = f32(pk_v[pl.ds(q + 48, 16)])
            labj = pk_v[pl.ds(q + 64, 16)]
            cx = (x1 + x2) * 0.5
            cy = (y1 + y2) * 0.5
            kx = (cx * inv_s - 0.5).astype(jnp.int32) - 1 + dx
            ky = (cy * inv_s - 0.5).astype(jnp.int32) - 1 + dy
            xw = (kx.astype(jnp.float32) + 0.5) * s
            yw = (ky.astype(jnp.float32) + 0.5) * s
            lft = xw - x1
            top = yw - y1
            rgt = x2 - xw
            bot = y2 - yw
            area = (lft + rgt) * (top + bot)
            omin = jnp.minimum(jnp.minimum(lft, top), jnp.minimum(rgt, bot))
            omax = jnp.maximum(jnp.maximum(lft, top), jnp.maximum(rgt, bot))
            cd = jnp.maximum(jnp.abs(xw - cx), jnp.abs(yw - cy))
            m = (omin > zero) & (omax > lo) & (omax <= hi) & (cd < rad)
            ploc = ky * h + kx + nb
            own = m & (ploc >= 0) & (ploc < size)
            idx = jnp.clip(ploc, 0, _BUF - 1)
            cur = plsc.load_gather(area_b, [idx], mask=own)
            upd = own & (area < cur)
            plsc.store_scatter(area_b, [idx], area, mask=upd)
            plsc.store_scatter(l_b, [idx], lft, mask=upd)
            plsc.store_scatter(t_b, [idx], top, mask=upd)
            plsc.store_scatter(r_b, [idx], rgt, mask=upd)
            plsc.store_scatter(b_b, [idx], bot, mask=upd)
            plsc.store_scatter(lab_b, [idx], labj, mask=upd)
            return carry

        @pl.when((lvl_hi > base) & (lvl_lo < base + size))
        def _():
            lax.fori_loop(0, _M, box_body, 0)

    neg1 = jnp.full((16,), -1.0, jnp.float32)
    izero = jnp.full((16,), 0, jnp.int32)
    thresh = jnp.full((16,), 9.0e7, jnp.float32)
    magic = jnp.full((16,), 0x1FBD1DF5, jnp.int32)

    def fin_body(i, carry):
        o = i * 16
        av = area_b[pl.ds(o, 16)]
        bl = l_b[pl.ds(o, 16)]
        bt = t_b[pl.ds(o, 16)]
        br = r_b[pl.ds(o, 16)]
        bb = b_b[pl.ds(o, 16)]
        blab = lab_b[pl.ds(o, 16)]
        posm = av < thresh
        lrmin = jnp.minimum(bl, br)
        lrmax = jnp.maximum(bl, br)
        tbmin = jnp.minimum(bt, bb)
        tbmax = jnp.maximum(bt, bb)
        ratio = lrmin * tbmin / (lrmax * tbmax + 1e-10)
        rs = jnp.where(posm, ratio, one)
        sq = lax.bitcast_convert_type(
            (lax.bitcast_convert_type(rs, jnp.int32) >> 1) + magic,
            jnp.float32)
        sq = (sq + rs / sq) * 0.5
        sq = (sq + rs / sq) * 0.5
        area_b[pl.ds(o, 16)] = jnp.where(posm, sq, neg1)
        lab_b[pl.ds(o, 16)] = jnp.where(posm, blab, izero)
        l_b[pl.ds(o, 16)] = jnp.where(posm, bl, neg1)
        t_b[pl.ds(o, 16)] = jnp.where(posm, bt, neg1)
        r_b[pl.ds(o, 16)] = jnp.where(posm, br, neg1)
        b_b[pl.ds(o, 16)] = jnp.where(posm, bb, neg1)
        return carry

    lax.fori_loop(0, nv, fin_body, 0)

    off = pl.multiple_of(bat * _P + base, 16)
    pairs = ((lab_b, cls_o), (area_b, cen_o), (l_b, l_o),
             (t_b, t_o), (r_b, r_o), (b_b, b_o))
    for sz, cond in ((3280, k < 4), (3264, k == 4),
                     ((2048), (k == 5) | (k == 6)), (1536, k == 7)):
        @pl.when(cond)
        def _(sz=sz):
            handles = [
                pltpu.async_copy(src.at[pl.ds(0, sz)],
                                 dst.at[pl.ds(off, sz)], sem)
                for src, dst in pairs
            ]
            for hd in handles:
                hd.wait()


def kernel(cls_logit_0, center_logit_0, reg_logit_0,
           cls_logit_1, center_logit_1, reg_logit_1,
           cls_logit_2, center_logit_2, reg_logit_2,
           cls_logit_3, center_logit_3, reg_logit_3,
           cls_logit_4, center_logit_4, reg_logit_4,
           gt_box, labels):
    packed = jnp.concatenate([
        lax.bitcast_convert_type(gt_box.astype(jnp.float32), jnp.int32),
        labels.astype(jnp.int32)[..., None],
    ], axis=-1)
    packed_bc = jnp.broadcast_to(packed[..., None], (_B, _M, 5, 16)).reshape(-1)
    cls_f, cen_f, l_f, t_f, r_f, b_f = _build_sc_targets()(packed_bc)
    cls_t = cls_f.reshape(_B, _P)[:, :_HW, None]
    cen_t = cen_f.reshape(_B, _P)[:, :_HW, None]
    reg_t = jnp.stack(
        [a.reshape(_B, _P)[:, :_HW] for a in (l_f, t_f, r_f, b_f)], axis=-1)
    return cls_t, cen_t, reg_t
